# Initial kernel scaffold; baseline (speedup 1.0000x reference)
import jax
import jax.numpy as jnp
from jax.experimental import pallas as pl


def kernel(h, pos, edge_index, edge_type, node_type, emb, W1, b1, W2, b2, U1, c1, U2, c2, gamma_ln, beta_ln):
    raise NotImplementedError("write your pallas kernel here")


# speedup vs baseline: 4.4827x; 4.4827x over previous
import functools

import jax
import jax.numpy as jnp
from jax import lax
from jax.experimental import pallas as pl
from jax.experimental.pallas import tpu as pltpu
from jax.experimental.pallas import tpu_sc as plsc

N = 10000
E = 320000
D = 128
NUM_RBF = 32
CUTOFF = 6.0
STEP = CUTOFF / (NUM_RBF - 1)
GAMMA = 1.0 / max(STEP * STEP, 1e-06)

PW = 8
CW = 16
NBLK = 400
NB = N // NBLK
EBLK = 2000
EB = E // EBLK
NC = 2
NS = 16
NW = NC * NS
EPW = E // NW
CH = 80
NCH = EPW // CH
RPT = N // NS


def _proj_body(h_ref, w1at_ref, w1bt_ref, w1ct_ref, emb8_ref,
               ha_ref, hb_ref, ec_ref):
    h = h_ref[...]
    ha_ref[...] = jnp.dot(h, w1at_ref[...], preferred_element_type=jnp.float32)
    hb_ref[...] = jnp.dot(h, w1bt_ref[...], preferred_element_type=jnp.float32)
    ec_ref[...] = jnp.dot(emb8_ref[...], w1ct_ref[...],
                          preferred_element_type=jnp.float32)


_proj_call = pl.pallas_call(
    _proj_body,
    grid=(NB,),
    in_specs=[
        pl.BlockSpec((NBLK, D), lambda i: (i, 0)),
        pl.BlockSpec((D, D), lambda i: (0, 0)),
        pl.BlockSpec((D, D), lambda i: (0, 0)),
        pl.BlockSpec((D, D), lambda i: (0, 0)),
        pl.BlockSpec((8, D), lambda i: (0, 0)),
    ],
    out_specs=[
        pl.BlockSpec((NBLK, D), lambda i: (i, 0)),
        pl.BlockSpec((NBLK, D), lambda i: (i, 0)),
        pl.BlockSpec((8, D), lambda i: (0, 0)),
    ],
    out_shape=[
        jax.ShapeDtypeStruct((N, D), jnp.float32),
        jax.ShapeDtypeStruct((N, D), jnp.float32),
        jax.ShapeDtypeStruct((8, D), jnp.float32),
    ],
)


@functools.cache
def _sc_mesh():
    return plsc.VectorSubcoreMesh(core_axis_name="c", subcore_axis_name="s")


@functools.cache
def _edge_gather_call():
    return pl.kernel(
        _edge_gather,
        out_type=(jax.ShapeDtypeStruct((E, D), jnp.float32),
                  jax.ShapeDtypeStruct((E, PW), jnp.float32),
                  jax.ShapeDtypeStruct((E, PW), jnp.float32)),
        mesh=_sc_mesh(),
        compiler_params=pltpu.CompilerParams(use_tc_tiling_on_sc=False),
        scratch_types=[
            pltpu.VMEM((CH,), jnp.int32),
            pltpu.VMEM((CH,), jnp.int32),
            pltpu.VMEM((CH, D), jnp.float32),
            pltpu.VMEM((CH, D), jnp.float32),
            pltpu.VMEM((CH, PW), jnp.float32),
            pltpu.VMEM((CH, PW), jnp.float32),
            pltpu.SemaphoreType.DMA,
            pltpu.SemaphoreType.DMA,
            pltpu.SemaphoreType.DMA,
            pltpu.SemaphoreType.DMA,
        ],
    )


def _edge_gather(src_hbm, dst_hbm, tha_hbm, thb_hbm, pos8_hbm,
                 g_hbm, ga_hbm, gb_hbm,
                 si_v, di_v, ra_v, rb_v, pa_v, pb_v,
                 sem_a, sem_b, sem_c, sem_d):
    wid = lax.axis_index("c") * NS + lax.axis_index("s")
    base = wid * EPW

    def chunk(j, carry):
        off = base + j * CH
        pltpu.sync_copy(src_hbm.at[pl.ds(off, CH)], si_v)
        pltpu.sync_copy(dst_hbm.at[pl.ds(off, CH)], di_v)
        cp_a = pltpu.async_copy(tha_hbm.at[si_v], ra_v, sem_a)
        cp_b = pltpu.async_copy(thb_hbm.at[di_v], rb_v, sem_b)
        cp_c = pltpu.async_copy(pos8_hbm.at[si_v], pa_v, sem_c)
        cp_d = pltpu.async_copy(pos8_hbm.at[di_v], pb_v, sem_d)
        cp_a.wait()
        cp_b.wait()

        def add_row(r, c2):
            for k in range(D // 16):
                sl = pl.ds(k * 16, 16)
                ra_v[r, sl] = ra_v[r, sl] + rb_v[r, sl]
            return c2

        lax.fori_loop(0, CH, add_row, 0)
        cp_c.wait()
        cp_d.wait()
        pltpu.sync_copy(ra_v, g_hbm.at[pl.ds(off, CH)])
        pltpu.sync_copy(pa_v, ga_hbm.at[pl.ds(off, CH)])
        pltpu.sync_copy(pb_v, gb_hbm.at[pl.ds(off, CH)])
        return carry

    lax.fori_loop(0, NCH, chunk, 0)


def _edge_mlp_body(g_ref, ga_ref, gb_ref, etf_ref, wr_ref, w2t_ref, prm_ref,
                   out_ref):
    base = g_ref[...]
    rel = ga_ref[...] - gb_ref[...]
    t = etf_ref[...]
    dist2 = jnp.sum(rel * rel, axis=1, keepdims=True)
    dist = jnp.sqrt(dist2)
    centers = lax.broadcasted_iota(
        jnp.int32, (1, NUM_RBF), 1).astype(jnp.float32) * STEP
    diff = dist - centers
    radial = jnp.exp(-GAMMA * diff * diff)
    prm = prm_ref[...]
    bias = prm[0:1, :]
    wd = prm[1:2, :]
    we = prm[2:3, :]
    b2 = prm[3:4, :]
    pre = (base + bias + dist * wd + t * we
           + jnp.dot(radial, wr_ref[...], preferred_element_type=jnp.float32))
    x = pre * jax.nn.sigmoid(pre)
    m = jnp.dot(x, w2t_ref[...], preferred_element_type=jnp.float32) + b2
    out_ref[...] = m * jax.nn.sigmoid(m)


_edge_mlp_call = pl.pallas_call(
    _edge_mlp_body,
    grid=(EB,),
    in_specs=[
        pl.BlockSpec((EBLK, D), lambda i: (i, 0)),
        pl.BlockSpec((EBLK, PW), lambda i: (i, 0)),
        pl.BlockSpec((EBLK, PW), lambda i: (i, 0)),
        pl.BlockSpec((EBLK, 1), lambda i: (i, 0)),
        pl.BlockSpec((NUM_RBF, D), lambda i: (0, 0)),
        pl.BlockSpec((D, D), lambda i: (0, 0)),
        pl.BlockSpec((8, D), lambda i: (0, 0)),
    ],
    out_specs=pl.BlockSpec((EBLK, D), lambda i: (i, 0)),
    out_shape=jax.ShapeDtypeStruct((E, D), jnp.float32),
)


@functools.cache
def _edge_scatter_call():
    return pl.kernel(
        _edge_scatter,
        out_type=(jax.ShapeDtypeStruct((NC, N, D), jnp.float32),
                  jax.ShapeDtypeStruct((NC, N, CW), jnp.float32)),
        mesh=_sc_mesh(),
        compiler_params=pltpu.CompilerParams(use_tc_tiling_on_sc=False),
        scratch_types=[
            pltpu.VMEM((CH,), jnp.int32),
            pltpu.VMEM((CH, D), jnp.float32),
            pltpu.VMEM((CH, CW), jnp.float32),
            pltpu.VMEM_SHARED((N, D), jnp.float32),
            pltpu.VMEM_SHARED((N, CW), jnp.float32),
        ],
    )


def _edge_scatter(dst_hbm, msg_hbm, zmsg_hbm, zcnt_hbm, omsg_hbm, ocnt_hbm,
                  di_v, buf_v, ones_v, acc_sh, cnt_sh):
    cid = lax.axis_index("c")
    sid = lax.axis_index("s")
    wid = cid * NS + sid
    rbase = sid * RPT
    pltpu.sync_copy(zmsg_hbm.at[pl.ds(rbase, RPT)],
                    acc_sh.at[pl.ds(rbase, RPT)])
    pltpu.sync_copy(zcnt_hbm.at[pl.ds(rbase, RPT)],
                    cnt_sh.at[pl.ds(rbase, RPT)])
    onerow = jnp.where(lax.iota(jnp.int32, CW) == 0, 1.0, 0.0)

    def set_row(r, c):
        ones_v[r] = onerow
        return c

    lax.fori_loop(0, CH, set_row, 0)
    plsc.subcore_barrier()
    base = wid * EPW

    def chunk(j, carry):
        off = base + j * CH
        pltpu.sync_copy(dst_hbm.at[pl.ds(off, CH)], di_v)
        pltpu.sync_copy(msg_hbm.at[pl.ds(off, CH)], buf_v)
        pltpu.sync_copy(buf_v, acc_sh.at[di_v], add=True)
        pltpu.sync_copy(ones_v, cnt_sh.at[di_v], add=True)
        return carry

    lax.fori_loop(0, NCH, chunk, 0)
    plsc.subcore_barrier()
    pltpu.sync_copy(acc_sh.at[pl.ds(rbase, RPT)],
                    omsg_hbm.at[cid, pl.ds(rbase, RPT)])
    pltpu.sync_copy(cnt_sh.at[pl.ds(rbase, RPT)],
                    ocnt_hbm.at[cid, pl.ds(rbase, RPT)])


def _node_body(p0_ref, p1_ref, c0_ref, c1_ref, h_ref, ntf_ref,
               u1ht_ref, u1at_ref, u2t_ref, prm_ref, out_ref):
    cnt = jnp.maximum(c0_ref[:, :1] + c1_ref[:, :1], 1.0)
    agg = (p0_ref[...] + p1_ref[...]) / cnt
    h = h_ref[...]
    prm = prm_ref[...]
    c1row = prm[0:1, :]
    c2row = prm[1:2, :]
    gam = prm[2:3, :]
    bet = prm[3:4, :]
    upre = (jnp.dot(h, u1ht_ref[...], preferred_element_type=jnp.float32)
            + jnp.dot(agg, u1at_ref[...], preferred_element_type=jnp.float32)
            + c1row)
    u = upre * jax.nn.sigmoid(upre)
    upd = jnp.dot(u, u2t_ref[...], preferred_element_type=jnp.float32) + c2row
    pre = h + upd
    mu = jnp.mean(pre, axis=1, keepdims=True)
    cen = pre - mu
    var = jnp.mean(cen * cen, axis=1, keepdims=True)
    ln = cen / jnp.sqrt(var + 1e-05) * gam + bet
    out_ref[...] = jnp.where(ntf_ref[...] == 1.0, ln, h)


_node_call = pl.pallas_call(
    _node_body,
    grid=(NB,),
    in_specs=[
        pl.BlockSpec((NBLK, D), lambda i: (i, 0)),
        pl.BlockSpec((NBLK, D), lambda i: (i, 0)),
        pl.BlockSpec((NBLK, CW), lambda i: (i, 0)),
        pl.BlockSpec((NBLK, CW), lambda i: (i, 0)),
        pl.BlockSpec((NBLK, D), lambda i: (i, 0)),
        pl.BlockSpec((NBLK, 1), lambda i: (i, 0)),
        pl.BlockSpec((D, D), lambda i: (0, 0)),
        pl.BlockSpec((D, D), lambda i: (0, 0)),
        pl.BlockSpec((D, D), lambda i: (0, 0)),
        pl.BlockSpec((8, D), lambda i: (0, 0)),
    ],
    out_specs=pl.BlockSpec((NBLK, D), lambda i: (i, 0)),
    out_shape=jax.ShapeDtypeStruct((N, D), jnp.float32),
)


def kernel(h, pos, edge_index, edge_type, node_type, emb, W1, b1, W2, b2,
           U1, c1, U2, c2, gamma_ln, beta_ln):
    src = edge_index[0]
    dst = edge_index[1]
    etf = edge_type.astype(jnp.float32).reshape(E, 1)
    ntf = node_type.astype(jnp.float32).reshape(N, 1)

    w1at = W1[:, :D].T
    w1bt = W1[:, D:2 * D].T
    w1ct = W1[:, 2 * D:3 * D].T
    wr = W1[:, 3 * D:3 * D + NUM_RBF].T
    wd = W1[:, 3 * D + NUM_RBF]
    emb8 = jnp.concatenate([emb, jnp.zeros((8 - emb.shape[0], D), jnp.float32)])

    ha, hb, ec28 = _proj_call(h, w1at, w1bt, w1ct, emb8)
    ec0 = ec28[0]
    dec = ec28[1] - ec28[0]
    prm_c = jnp.concatenate([
        jnp.stack([b1 + ec0, wd, dec, b2]),
        jnp.zeros((4, D), jnp.float32),
    ])
    pos8 = jnp.concatenate([pos, jnp.zeros((N, PW - 3), jnp.float32)], axis=1)

    g, ga, gb = _edge_gather_call()(src, dst, ha, hb, pos8)
    msg = _edge_mlp_call(g, ga, gb, etf, wr, W2.T, prm_c)

    zmsg = jnp.zeros((N, D), jnp.float32)
    zcnt = jnp.zeros((N, CW), jnp.float32)
    omsg, ocnt = _edge_scatter_call()(dst, msg, zmsg, zcnt)

    prm_e = jnp.concatenate([
        jnp.stack([c1, c2, gamma_ln, beta_ln]),
        jnp.zeros((4, D), jnp.float32),
    ])
    return _node_call(omsg[0], omsg[1], ocnt[0], ocnt[1], h, ntf,
                      U1[:, :D].T, U1[:, D:].T, U2.T, prm_e)


# speedup vs baseline: 4.7229x; 1.0536x over previous
import functools

import jax
import jax.numpy as jnp
from jax import lax
from jax.experimental import pallas as pl
from jax.experimental.pallas import tpu as pltpu
from jax.experimental.pallas import tpu_sc as plsc

N = 10000
E = 320000
D = 128
NUM_RBF = 32
CUTOFF = 6.0
STEP = CUTOFF / (NUM_RBF - 1)
GAMMA = 1.0 / max(STEP * STEP, 1e-06)

PW = 8
CW = 16
NBLK = 400
NB = N // NBLK
EBLK = 2000
EB = E // EBLK
NC = 2
NS = 16
NW = NC * NS
EPW = E // NW
CH = 80
NCH = EPW // CH
RPT = N // NS


def _proj_body(h_ref, w1at_ref, w1bt_ref, w1ct_ref, emb8_ref,
               ha_ref, hb_ref, ec_ref):
    h = h_ref[...]
    ha_ref[...] = jnp.dot(
        h, w1at_ref[...], preferred_element_type=jnp.float32
    ).astype(jnp.bfloat16)
    hb_ref[...] = jnp.dot(
        h, w1bt_ref[...], preferred_element_type=jnp.float32
    ).astype(jnp.bfloat16)
    ec_ref[...] = jnp.dot(emb8_ref[...], w1ct_ref[...],
                          preferred_element_type=jnp.float32)


_proj_call = pl.pallas_call(
    _proj_body,
    grid=(NB,),
    in_specs=[
        pl.BlockSpec((NBLK, D), lambda i: (i, 0)),
        pl.BlockSpec((D, D), lambda i: (0, 0)),
        pl.BlockSpec((D, D), lambda i: (0, 0)),
        pl.BlockSpec((D, D), lambda i: (0, 0)),
        pl.BlockSpec((8, D), lambda i: (0, 0)),
    ],
    out_specs=[
        pl.BlockSpec((NBLK, D), lambda i: (i, 0)),
        pl.BlockSpec((NBLK, D), lambda i: (i, 0)),
        pl.BlockSpec((8, D), lambda i: (0, 0)),
    ],
    out_shape=[
        jax.ShapeDtypeStruct((N, D), jnp.bfloat16),
        jax.ShapeDtypeStruct((N, D), jnp.bfloat16),
        jax.ShapeDtypeStruct((8, D), jnp.float32),
    ],
)


@functools.cache
def _sc_mesh():
    return plsc.VectorSubcoreMesh(core_axis_name="c", subcore_axis_name="s")


NSETS = 5
NBODY = NCH // NSETS
DCH = 40
DSETS = 2
DNCH = EPW // DCH
DNBODY = DNCH // DSETS


@functools.cache
def _edge_gather_call():
    per_set = [
        pltpu.VMEM((CH,), jnp.int32),
        pltpu.VMEM((CH,), jnp.int32),
        pltpu.VMEM((CH, D), jnp.bfloat16),
        pltpu.VMEM((CH, D), jnp.bfloat16),
        pltpu.VMEM((CH, PW), jnp.float32),
        pltpu.VMEM((CH, PW), jnp.float32),
        pltpu.SemaphoreType.DMA,
    ]
    return pl.kernel(
        _edge_gather,
        out_type=(jax.ShapeDtypeStruct((E, D), jnp.bfloat16),
                  jax.ShapeDtypeStruct((E, PW), jnp.float32),
                  jax.ShapeDtypeStruct((E, PW), jnp.float32)),
        mesh=_sc_mesh(),
        compiler_params=pltpu.CompilerParams(use_tc_tiling_on_sc=False),
        scratch_types=per_set * NSETS + [
            pltpu.SemaphoreType.DMA,
            pltpu.SemaphoreType.DMA,
        ],
    )


def _edge_gather(src_hbm, dst_hbm, tha_hbm, thb_hbm, pos8_hbm,
                 g_hbm, ga_hbm, gb_hbm, *scr):
    sets = [scr[7 * s:7 * s + 7] for s in range(NSETS)]
    sem_i = scr[7 * NSETS]
    sem_o = scr[7 * NSETS + 1]
    wid = lax.axis_index("c") * NS + lax.axis_index("s")
    base = wid * EPW

    def drain_outs():
        for si_v, di_v, ra_v, rb_v, pa_v, pb_v, sem_g in sets:
            pltpu.make_async_copy(g_hbm.at[pl.ds(0, CH)], ra_v, sem_o).wait()
            pltpu.make_async_copy(ga_hbm.at[pl.ds(0, CH)], pa_v, sem_o).wait()
            pltpu.make_async_copy(gb_hbm.at[pl.ds(0, CH)], pb_v, sem_o).wait()

    def body(j, carry):
        @pl.when(j > 0)
        def _():
            drain_outs()

        offs = [base + (j * NSETS + s) * CH for s in range(NSETS)]
        idx_cps = []
        for off, (si_v, di_v, *_rest) in zip(offs, sets):
            idx_cps.append(
                pltpu.async_copy(src_hbm.at[pl.ds(off, CH)], si_v, sem_i))
            idx_cps.append(
                pltpu.async_copy(dst_hbm.at[pl.ds(off, CH)], di_v, sem_i))
        for cp in idx_cps:
            cp.wait()
        gather_cps = []
        for si_v, di_v, ra_v, rb_v, pa_v, pb_v, sem_g in sets:
            gather_cps.append((
                pltpu.async_copy(tha_hbm.at[si_v], ra_v, sem_g),
                pltpu.async_copy(thb_hbm.at[di_v], rb_v, sem_g),
                pltpu.async_copy(pos8_hbm.at[si_v], pa_v, sem_g),
                pltpu.async_copy(pos8_hbm.at[di_v], pb_v, sem_g),
            ))
        for off, (si_v, di_v, ra_v, rb_v, pa_v, pb_v, sem_g), cps in zip(
                offs, sets, gather_cps):
            for cp in cps:
                cp.wait()

            def add4(r, c2, ra_v=ra_v, rb_v=rb_v):
                for rr in range(4):
                    row = r * 4 + rr
                    for k in range(D // 32):
                        sl = pl.ds(k * 32, 32)
                        ra_v[row, sl] = ra_v[row, sl] + rb_v[row, sl]
                return c2

            lax.fori_loop(0, CH // 4, add4, 0)
            pltpu.async_copy(ra_v, g_hbm.at[pl.ds(off, CH)], sem_o)
            pltpu.async_copy(pa_v, ga_hbm.at[pl.ds(off, CH)], sem_o)
            pltpu.async_copy(pb_v, gb_hbm.at[pl.ds(off, CH)], sem_o)
        return carry

    lax.fori_loop(0, NBODY, body, 0)
    drain_outs()


def _edge_mlp_body(g_ref, ga_ref, gb_ref, etf_ref, wr_ref, w2t_ref, prm_ref,
                   out_ref):
    base = g_ref[...].astype(jnp.float32)
    rel = ga_ref[...] - gb_ref[...]
    t = etf_ref[...]
    dist2 = jnp.sum(rel * rel, axis=1, keepdims=True)
    dist = jnp.sqrt(dist2)
    centers = lax.broadcasted_iota(
        jnp.int32, (1, NUM_RBF), 1).astype(jnp.float32) * STEP
    diff = dist - centers
    radial = jnp.exp(-GAMMA * diff * diff)
    prm = prm_ref[...]
    bias = prm[0:1, :]
    wd = prm[1:2, :]
    we = prm[2:3, :]
    b2 = prm[3:4, :]
    pre = (base + bias + dist * wd + t * we
           + jnp.dot(radial, wr_ref[...], preferred_element_type=jnp.float32))
    x = pre * jax.nn.sigmoid(pre)
    m = jnp.dot(x, w2t_ref[...], preferred_element_type=jnp.float32) + b2
    out_ref[...] = m * jax.nn.sigmoid(m)


_edge_mlp_call = pl.pallas_call(
    _edge_mlp_body,
    grid=(EB,),
    in_specs=[
        pl.BlockSpec((EBLK, D), lambda i: (i, 0)),
        pl.BlockSpec((EBLK, PW), lambda i: (i, 0)),
        pl.BlockSpec((EBLK, PW), lambda i: (i, 0)),
        pl.BlockSpec((EBLK, 1), lambda i: (i, 0)),
        pl.BlockSpec((NUM_RBF, D), lambda i: (0, 0)),
        pl.BlockSpec((D, D), lambda i: (0, 0)),
        pl.BlockSpec((8, D), lambda i: (0, 0)),
    ],
    out_specs=pl.BlockSpec((EBLK, D), lambda i: (i, 0)),
    out_shape=jax.ShapeDtypeStruct((E, D), jnp.float32),
)


@functools.cache
def _edge_scatter_call():
    return pl.kernel(
        _edge_scatter,
        out_type=(jax.ShapeDtypeStruct((NC, N, D), jnp.float32),
                  jax.ShapeDtypeStruct((NC, N, CW), jnp.float32)),
        mesh=_sc_mesh(),
        compiler_params=pltpu.CompilerParams(use_tc_tiling_on_sc=False),
        scratch_types=[
            pltpu.VMEM((DCH,), jnp.int32),
            pltpu.VMEM((DCH, D), jnp.float32),
            pltpu.SemaphoreType.DMA,
        ] * DSETS + [
            pltpu.VMEM((DCH, CW), jnp.float32),
            pltpu.VMEM_SHARED((N, D), jnp.float32),
            pltpu.VMEM_SHARED((N, CW), jnp.float32),
            pltpu.SemaphoreType.DMA,
        ],
    )


def _edge_scatter(dst_hbm, msg_hbm, zmsg_hbm, zcnt_hbm, omsg_hbm, ocnt_hbm,
                  *scr):
    sets = [scr[3 * s:3 * s + 3] for s in range(DSETS)]
    ones_v, acc_sh, cnt_sh, sem_i = scr[3 * DSETS:]
    cid = lax.axis_index("c")
    sid = lax.axis_index("s")
    wid = cid * NS + sid
    rbase = sid * RPT
    pltpu.sync_copy(zmsg_hbm.at[pl.ds(rbase, RPT)],
                    acc_sh.at[pl.ds(rbase, RPT)])
    pltpu.sync_copy(zcnt_hbm.at[pl.ds(rbase, RPT)],
                    cnt_sh.at[pl.ds(rbase, RPT)])
    onerow = jnp.where(lax.iota(jnp.int32, CW) == 0, 1.0, 0.0)

    def set_row(r, c):
        ones_v[r] = onerow
        return c

    lax.fori_loop(0, DCH, set_row, 0)
    plsc.subcore_barrier()
    base = wid * EPW

    def drain_set(di_v, buf_v, sem_sc):
        pltpu.make_async_copy(msg_hbm.at[pl.ds(0, DCH)], buf_v, sem_sc).wait()
        pltpu.make_async_copy(zcnt_hbm.at[pl.ds(0, DCH)], ones_v,
                              sem_sc).wait()

    def body(j, carry):
        offs = [base + (j * DSETS + s) * DCH for s in range(DSETS)]
        cps = []
        for off, (di_v, buf_v, sem_sc) in zip(offs, sets):
            @pl.when(j > 0)
            def _(di_v=di_v, buf_v=buf_v, sem_sc=sem_sc):
                drain_set(di_v, buf_v, sem_sc)

            cps.append((
                pltpu.async_copy(dst_hbm.at[pl.ds(off, DCH)], di_v, sem_i),
                pltpu.async_copy(msg_hbm.at[pl.ds(off, DCH)], buf_v, sem_i),
            ))
        for (cp_i, cp_m), (di_v, buf_v, sem_sc) in zip(cps, sets):
            cp_i.wait()
            cp_m.wait()
            pltpu.async_copy(buf_v, acc_sh.at[di_v], sem_sc, add=True)
            pltpu.async_copy(ones_v, cnt_sh.at[di_v], sem_sc, add=True)
        return carry

    lax.fori_loop(0, DNBODY, body, 0)
    for di_v, buf_v, sem_sc in sets:
        drain_set(di_v, buf_v, sem_sc)
    plsc.subcore_barrier()
    pltpu.sync_copy(acc_sh.at[pl.ds(rbase, RPT)],
                    omsg_hbm.at[cid, pl.ds(rbase, RPT)])
    pltpu.sync_copy(cnt_sh.at[pl.ds(rbase, RPT)],
                    ocnt_hbm.at[cid, pl.ds(rbase, RPT)])


def _node_body(p0_ref, p1_ref, c0_ref, c1_ref, h_ref, ntf_ref,
               u1ht_ref, u1at_ref, u2t_ref, prm_ref, out_ref):
    cnt = jnp.maximum(c0_ref[:, :1] + c1_ref[:, :1], 1.0)
    agg = (p0_ref[...] + p1_ref[...]) / cnt
    h = h_ref[...]
    prm = prm_ref[...]
    c1row = prm[0:1, :]
    c2row = prm[1:2, :]
    gam = prm[2:3, :]
    bet = prm[3:4, :]
    upre = (jnp.dot(h, u1ht_ref[...], preferred_element_type=jnp.float32)
            + jnp.dot(agg, u1at_ref[...], preferred_element_type=jnp.float32)
            + c1row)
    u = upre * jax.nn.sigmoid(upre)
    upd = jnp.dot(u, u2t_ref[...], preferred_element_type=jnp.float32) + c2row
    pre = h + upd
    mu = jnp.mean(pre, axis=1, keepdims=True)
    cen = pre - mu
    var = jnp.mean(cen * cen, axis=1, keepdims=True)
    ln = cen / jnp.sqrt(var + 1e-05) * gam + bet
    out_ref[...] = jnp.where(ntf_ref[...] == 1.0, ln, h)


_node_call = pl.pallas_call(
    _node_body,
    grid=(NB,),
    in_specs=[
        pl.BlockSpec((NBLK, D), lambda i: (i, 0)),
        pl.BlockSpec((NBLK, D), lambda i: (i, 0)),
        pl.BlockSpec((NBLK, CW), lambda i: (i, 0)),
        pl.BlockSpec((NBLK, CW), lambda i: (i, 0)),
        pl.BlockSpec((NBLK, D), lambda i: (i, 0)),
        pl.BlockSpec((NBLK, 1), lambda i: (i, 0)),
        pl.BlockSpec((D, D), lambda i: (0, 0)),
        pl.BlockSpec((D, D), lambda i: (0, 0)),
        pl.BlockSpec((D, D), lambda i: (0, 0)),
        pl.BlockSpec((8, D), lambda i: (0, 0)),
    ],
    out_specs=pl.BlockSpec((NBLK, D), lambda i: (i, 0)),
    out_shape=jax.ShapeDtypeStruct((N, D), jnp.float32),
)


def kernel(h, pos, edge_index, edge_type, node_type, emb, W1, b1, W2, b2,
           U1, c1, U2, c2, gamma_ln, beta_ln):
    src = edge_index[0]
    dst = edge_index[1]
    etf = edge_type.astype(jnp.float32).reshape(E, 1)
    ntf = node_type.astype(jnp.float32).reshape(N, 1)

    w1at = W1[:, :D].T
    w1bt = W1[:, D:2 * D].T
    w1ct = W1[:, 2 * D:3 * D].T
    wr = W1[:, 3 * D:3 * D + NUM_RBF].T
    wd = W1[:, 3 * D + NUM_RBF]
    emb8 = jnp.concatenate([emb, jnp.zeros((8 - emb.shape[0], D), jnp.float32)])

    ha, hb, ec28 = _proj_call(h, w1at, w1bt, w1ct, emb8)
    ec0 = ec28[0]
    dec = ec28[1] - ec28[0]
    prm_c = jnp.concatenate([
        jnp.stack([b1 + ec0, wd, dec, b2]),
        jnp.zeros((4, D), jnp.float32),
    ])
    pos8 = jnp.concatenate([pos, jnp.zeros((N, PW - 3), jnp.float32)], axis=1)

    g, ga, gb = _edge_gather_call()(src, dst, ha, hb, pos8)
    msg = _edge_mlp_call(g, ga, gb, etf, wr, W2.T, prm_c)

    zmsg = jnp.zeros((N, D), jnp.float32)
    zcnt = jnp.zeros((N, CW), jnp.float32)
    omsg, ocnt = _edge_scatter_call()(dst, msg, zmsg, zcnt)

    prm_e = jnp.concatenate([
        jnp.stack([c1, c2, gamma_ln, beta_ln]),
        jnp.zeros((4, D), jnp.float32),
    ])
    return _node_call(omsg[0], omsg[1], ocnt[0], ocnt[1], h, ntf,
                      U1[:, :D].T, U1[:, D:].T, U2.T, prm_e)


# speedup vs baseline: 6.0311x; 1.2770x over previous
import functools

import jax
import jax.numpy as jnp
from jax import lax
from jax.experimental import pallas as pl
from jax.experimental.pallas import tpu as pltpu
from jax.experimental.pallas import tpu_sc as plsc

N = 10000
E = 320000
D = 128
NUM_RBF = 32
CUTOFF = 6.0
STEP = CUTOFF / (NUM_RBF - 1)
GAMMA = 1.0 / max(STEP * STEP, 1e-06)

PW = 8
CW = 16
AW = 8
NBLK = 400
NB = N // NBLK
EBLK = 2560
EB = E // EBLK
NC = 2
NS = 16
NW = NC * NS
EPW = E // NW
CH = 80
NCH = EPW // CH
RPT = N // NS
STRIPE = 624

NSETS = 5
NBODY = NCH // NSETS
DCH = 40
DSETS = 2
DNCH = EPW // DCH
DNBODY = DNCH // DSETS


def _proj_body(h_ref, w1at_ref, w1bt_ref, w1ct_ref, emb8_ref,
               ha_ref, hb_ref, ec_ref):
    h = h_ref[...]
    ha_ref[...] = jnp.dot(h, w1at_ref[...], preferred_element_type=jnp.float32)
    hb_ref[...] = jnp.dot(h, w1bt_ref[...], preferred_element_type=jnp.float32)
    ec_ref[...] = jnp.dot(emb8_ref[...], w1ct_ref[...],
                          preferred_element_type=jnp.float32)


_proj_call = pl.pallas_call(
    _proj_body,
    grid=(NB,),
    in_specs=[
        pl.BlockSpec((NBLK, D), lambda i: (i, 0)),
        pl.BlockSpec((D, D), lambda i: (0, 0)),
        pl.BlockSpec((D, D), lambda i: (0, 0)),
        pl.BlockSpec((D, D), lambda i: (0, 0)),
        pl.BlockSpec((8, D), lambda i: (0, 0)),
    ],
    out_specs=[
        pl.BlockSpec((NBLK, D), lambda i: (i, 0)),
        pl.BlockSpec((NBLK, D), lambda i: (i, 0)),
        pl.BlockSpec((8, D), lambda i: (0, 0)),
    ],
    out_shape=[
        jax.ShapeDtypeStruct((N, D), jnp.float32),
        jax.ShapeDtypeStruct((N, D), jnp.float32),
        jax.ShapeDtypeStruct((8, D), jnp.float32),
    ],
)


@functools.cache
def _sc_mesh():
    return plsc.VectorSubcoreMesh(core_axis_name="c", subcore_axis_name="s")


@functools.cache
def _edge_gather_call():
    per_set = [
        pltpu.VMEM((CH,), jnp.int32),
        pltpu.VMEM((CH,), jnp.int32),
        pltpu.VMEM((CH, D), jnp.float32),
        pltpu.VMEM((CH, D), jnp.float32),
        pltpu.SemaphoreType.DMA,
    ]
    return pl.kernel(
        _edge_gather,
        out_type=jax.ShapeDtypeStruct((E, D), jnp.float32),
        mesh=_sc_mesh(),
        scratch_types=per_set * NSETS + [
            pltpu.SemaphoreType.DMA,
            pltpu.SemaphoreType.DMA,
        ],
    )


def _edge_gather(src_hbm, dst_hbm, tha_hbm, thb_hbm, g_hbm, *scr):
    sets = [scr[5 * s:5 * s + 5] for s in range(NSETS)]
    sem_i = scr[5 * NSETS]
    sem_o = scr[5 * NSETS + 1]
    wid = lax.axis_index("c") * NS + lax.axis_index("s")
    base = wid * EPW

    def drain_outs():
        for si_v, di_v, ra_v, rb_v, sem_g in sets:
            pltpu.make_async_copy(g_hbm.at[pl.ds(0, CH)], ra_v, sem_o).wait()

    def body(j, carry):
        @pl.when(j > 0)
        def _():
            drain_outs()

        offs = [pl.multiple_of(base + (j * NSETS + s) * CH, 8)
                for s in range(NSETS)]
        idx_cps = []
        for off, (si_v, di_v, *_rest) in zip(offs, sets):
            idx_cps.append(
                pltpu.async_copy(src_hbm.at[pl.ds(off, CH)], si_v, sem_i))
            idx_cps.append(
                pltpu.async_copy(dst_hbm.at[pl.ds(off, CH)], di_v, sem_i))
        for cp in idx_cps:
            cp.wait()
        gather_cps = []
        for si_v, di_v, ra_v, rb_v, sem_g in sets:
            gather_cps.append((
                pltpu.async_copy(tha_hbm.at[si_v], ra_v, sem_g),
                pltpu.async_copy(thb_hbm.at[di_v], rb_v, sem_g),
            ))
        for off, (si_v, di_v, ra_v, rb_v, sem_g), cps in zip(
                offs, sets, gather_cps):
            for cp in cps:
                cp.wait()

            def add4(r, c2, ra_v=ra_v, rb_v=rb_v):
                for rr in range(4):
                    row = r * 4 + rr
                    for k in range(D // 16):
                        sl = pl.ds(k * 16, 16)
                        ra_v[row, sl] = ra_v[row, sl] + rb_v[row, sl]
                return c2

            lax.fori_loop(0, CH // 4, add4, 0)
            pltpu.async_copy(ra_v, g_hbm.at[pl.ds(off, CH)], sem_o)
        return carry

    lax.fori_loop(0, NBODY, body, 0)
    drain_outs()


@functools.cache
def _aux_call():
    return pl.kernel(
        _aux_kernel,
        out_type=(jax.ShapeDtypeStruct((AW, E), jnp.float32),
                  jax.ShapeDtypeStruct((NC, N, CW), jnp.float32)),
        mesh=_sc_mesh(),
        compiler_params=pltpu.CompilerParams(use_tc_tiling_on_sc=False,
                                             needs_layout_passes=False),
        scratch_types=[
            pltpu.VMEM((CH,), jnp.int32),
            pltpu.VMEM((CH,), jnp.int32),
            pltpu.VMEM((CH,), jnp.float32),
            pltpu.VMEM((CH, PW), jnp.float32),
            pltpu.VMEM((CH, PW), jnp.float32),
            pltpu.VMEM((AW, CH), jnp.float32),
            pltpu.VMEM((CH, CW), jnp.float32),
            pltpu.VMEM_SHARED((N, CW), jnp.float32),
            pltpu.SemaphoreType.DMA,
        ],
    )


def _aux_kernel(src_hbm, dst_hbm, etf_hbm, pos8_hbm, zcnt_hbm,
                relt_hbm, ocnt_hbm,
                si_v, di_v, et_v, pa_v, pb_v, rt_v, ones_v, cnt_sh, sem):
    cid = lax.axis_index("c")
    sid = lax.axis_index("s")
    wid = cid * NS + sid
    rbase = sid * RPT
    pltpu.sync_copy(zcnt_hbm.at[pl.ds(rbase, RPT)],
                    cnt_sh.at[pl.ds(rbase, RPT)])
    onerow = jnp.where(lax.iota(jnp.int32, CW) == 0, 1.0, 0.0)

    def set_row(r, c):
        ones_v[r] = onerow
        return c

    lax.fori_loop(0, CH, set_row, 0)
    plsc.subcore_barrier()
    base = wid * EPW

    def chunk(j, carry):
        off = base + j * CH
        pltpu.sync_copy(src_hbm.at[pl.ds(off, CH)], si_v)
        pltpu.sync_copy(dst_hbm.at[pl.ds(off, CH)], di_v)
        pltpu.sync_copy(etf_hbm.at[pl.ds(off, CH)], et_v)
        cp_a = pltpu.async_copy(pos8_hbm.at[si_v], pa_v, sem)
        cp_b = pltpu.async_copy(pos8_hbm.at[di_v], pb_v, sem)
        cp_a.wait()
        cp_b.wait()
        for k in range(CH // 16):
            sl = pl.ds(16 * k, 16)
            rows = lax.iota(jnp.int32, 16) + (16 * k)
            for c in range(3):
                cols = jnp.full((16,), c, jnp.int32)
                va = plsc.load_gather(pa_v, [rows, cols])
                vb = plsc.load_gather(pb_v, [rows, cols])
                rt_v[c, sl] = va - vb
            rt_v[3, sl] = et_v[sl]
        pltpu.sync_copy(rt_v, relt_hbm.at[:, pl.ds(off, CH)])
        pltpu.sync_copy(ones_v, cnt_sh.at[di_v], add=True)
        return carry

    lax.fori_loop(0, NCH, chunk, 0)
    plsc.subcore_barrier()
    pltpu.sync_copy(cnt_sh.at[pl.ds(rbase, RPT)],
                    ocnt_hbm.at[cid, pl.ds(rbase, RPT)])


def _edge_mlp_body(g_ref, relt_ref, wr_ref, w2t_ref, prm_ref, out_ref):
    base = g_ref[...]
    aux = jnp.transpose(relt_ref[...])
    rel = aux[:, 0:3]
    t = aux[:, 3:4]
    dist2 = jnp.sum(rel * rel, axis=1, keepdims=True)
    dist = jnp.sqrt(dist2)
    centers = lax.broadcasted_iota(
        jnp.int32, (1, NUM_RBF), 1).astype(jnp.float32) * STEP
    diff = dist - centers
    radial = jnp.exp(-GAMMA * diff * diff)
    prm = prm_ref[...]
    bias = prm[0:1, :]
    wd = prm[1:2, :]
    we = prm[2:3, :]
    b2 = prm[3:4, :]
    pre = (base + bias + dist * wd + t * we
           + jnp.dot(radial, wr_ref[...], preferred_element_type=jnp.float32))
    x = pre * jax.nn.sigmoid(pre)
    m = jnp.dot(x, w2t_ref[...], preferred_element_type=jnp.float32) + b2
    out_ref[...] = m * jax.nn.sigmoid(m)


_edge_mlp_call = pl.pallas_call(
    _edge_mlp_body,
    grid=(EB,),
    in_specs=[
        pl.BlockSpec((EBLK, D), lambda i: (i, 0)),
        pl.BlockSpec((AW, EBLK), lambda i: (0, i)),
        pl.BlockSpec((NUM_RBF, D), lambda i: (0, 0)),
        pl.BlockSpec((D, D), lambda i: (0, 0)),
        pl.BlockSpec((8, D), lambda i: (0, 0)),
    ],
    out_specs=pl.BlockSpec((EBLK, D), lambda i: (i, 0)),
    out_shape=jax.ShapeDtypeStruct((E, D), jnp.float32),
)


@functools.cache
def _edge_scatter_call():
    return pl.kernel(
        _edge_scatter,
        out_type=jax.ShapeDtypeStruct((NC, N, D), jnp.float32),
        mesh=_sc_mesh(),
        scratch_types=[
            pltpu.VMEM((DCH,), jnp.int32),
            pltpu.VMEM((DCH, D), jnp.float32),
            pltpu.SemaphoreType.DMA,
        ] * DSETS + [
            pltpu.VMEM_SHARED((N, D), jnp.float32),
            pltpu.SemaphoreType.DMA,
        ],
    )


def _edge_scatter(dst_hbm, msg_hbm, zmsg_hbm, omsg_hbm, *scr):
    sets = [scr[3 * s:3 * s + 3] for s in range(DSETS)]
    acc_sh, sem_i = scr[3 * DSETS:]
    cid = lax.axis_index("c")
    sid = lax.axis_index("s")
    wid = cid * NS + sid
    rbase = pl.multiple_of(sid * STRIPE, 8)
    pltpu.sync_copy(zmsg_hbm.at[pl.ds(rbase, STRIPE)],
                    acc_sh.at[pl.ds(rbase, STRIPE)])

    @pl.when(sid == 0)
    def _():
        pltpu.sync_copy(zmsg_hbm.at[pl.ds(NS * STRIPE, N - NS * STRIPE)],
                        acc_sh.at[pl.ds(NS * STRIPE, N - NS * STRIPE)])

    plsc.subcore_barrier()
    base = wid * EPW

    def drain_set(buf_v, sem_sc):
        pltpu.make_async_copy(msg_hbm.at[pl.ds(0, DCH)], buf_v, sem_sc).wait()

    def body(j, carry):
        offs = [pl.multiple_of(base + (j * DSETS + s) * DCH, 8)
                for s in range(DSETS)]
        cps = []
        for off, (di_v, buf_v, sem_sc) in zip(offs, sets):
            @pl.when(j > 0)
            def _(buf_v=buf_v, sem_sc=sem_sc):
                drain_set(buf_v, sem_sc)

            cps.append((
                pltpu.async_copy(dst_hbm.at[pl.ds(off, DCH)], di_v, sem_i),
                pltpu.async_copy(msg_hbm.at[pl.ds(off, DCH)], buf_v, sem_i),
            ))
        for (cp_i, cp_m), (di_v, buf_v, sem_sc) in zip(cps, sets):
            cp_i.wait()
            cp_m.wait()
            pltpu.async_copy(buf_v, acc_sh.at[di_v], sem_sc, add=True)
        return carry

    lax.fori_loop(0, DNBODY, body, 0)
    for di_v, buf_v, sem_sc in sets:
        drain_set(buf_v, sem_sc)
    plsc.subcore_barrier()
    pltpu.sync_copy(acc_sh.at[pl.ds(rbase, STRIPE)],
                    omsg_hbm.at[cid, pl.ds(rbase, STRIPE)])

    @pl.when(sid == 0)
    def _():
        pltpu.sync_copy(acc_sh.at[pl.ds(NS * STRIPE, N - NS * STRIPE)],
                        omsg_hbm.at[cid, pl.ds(NS * STRIPE, N - NS * STRIPE)])


def _node_body(p0_ref, p1_ref, c0_ref, c1_ref, h_ref, ntf_ref,
               u1ht_ref, u1at_ref, u2t_ref, prm_ref, out_ref):
    cnt = jnp.maximum(c0_ref[:, :1] + c1_ref[:, :1], 1.0)
    agg = (p0_ref[...] + p1_ref[...]) / cnt
    h = h_ref[...]
    prm = prm_ref[...]
    c1row = prm[0:1, :]
    c2row = prm[1:2, :]
    gam = prm[2:3, :]
    bet = prm[3:4, :]
    upre = (jnp.dot(h, u1ht_ref[...], preferred_element_type=jnp.float32)
            + jnp.dot(agg, u1at_ref[...], preferred_element_type=jnp.float32)
            + c1row)
    u = upre * jax.nn.sigmoid(upre)
    upd = jnp.dot(u, u2t_ref[...], preferred_element_type=jnp.float32) + c2row
    pre = h + upd
    mu = jnp.mean(pre, axis=1, keepdims=True)
    cen = pre - mu
    var = jnp.mean(cen * cen, axis=1, keepdims=True)
    ln = cen / jnp.sqrt(var + 1e-05) * gam + bet
    out_ref[...] = jnp.where(ntf_ref[...] == 1.0, ln, h)


_node_call = pl.pallas_call(
    _node_body,
    grid=(NB,),
    in_specs=[
        pl.BlockSpec((NBLK, D), lambda i: (i, 0)),
        pl.BlockSpec((NBLK, D), lambda i: (i, 0)),
        pl.BlockSpec((NBLK, CW), lambda i: (i, 0)),
        pl.BlockSpec((NBLK, CW), lambda i: (i, 0)),
        pl.BlockSpec((NBLK, D), lambda i: (i, 0)),
        pl.BlockSpec((NBLK, 1), lambda i: (i, 0)),
        pl.BlockSpec((D, D), lambda i: (0, 0)),
        pl.BlockSpec((D, D), lambda i: (0, 0)),
        pl.BlockSpec((D, D), lambda i: (0, 0)),
        pl.BlockSpec((8, D), lambda i: (0, 0)),
    ],
    out_specs=pl.BlockSpec((NBLK, D), lambda i: (i, 0)),
    out_shape=jax.ShapeDtypeStruct((N, D), jnp.float32),
)


def kernel(h, pos, edge_index, edge_type, node_type, emb, W1, b1, W2, b2,
           U1, c1, U2, c2, gamma_ln, beta_ln):
    src = edge_index[0]
    dst = edge_index[1]
    etf = edge_type.astype(jnp.float32)
    ntf = node_type.astype(jnp.float32).reshape(N, 1)

    w1at = W1[:, :D].T
    w1bt = W1[:, D:2 * D].T
    w1ct = W1[:, 2 * D:3 * D].T
    wr = W1[:, 3 * D:3 * D + NUM_RBF].T
    wd = W1[:, 3 * D + NUM_RBF]
    emb8 = jnp.concatenate([emb, jnp.zeros((8 - emb.shape[0], D), jnp.float32)])

    ha, hb, ec28 = _proj_call(h, w1at, w1bt, w1ct, emb8)
    ec0 = ec28[0]
    dec = ec28[1] - ec28[0]
    prm_c = jnp.concatenate([
        jnp.stack([b1 + ec0, wd, dec, b2]),
        jnp.zeros((4, D), jnp.float32),
    ])
    pos8 = jnp.concatenate([pos, jnp.zeros((N, PW - 3), jnp.float32)], axis=1)

    g = _edge_gather_call()(src, dst, ha, hb)
    zcnt = jnp.zeros((N, CW), jnp.float32)
    relt, ocnt = _aux_call()(src, dst, etf, pos8, zcnt)
    msg = _edge_mlp_call(g, relt, wr, W2.T, prm_c)

    zmsg = jnp.zeros((N, D), jnp.float32)
    omsg = _edge_scatter_call()(dst, msg, zmsg)

    prm_e = jnp.concatenate([
        jnp.stack([c1, c2, gamma_ln, beta_ln]),
        jnp.zeros((4, D), jnp.float32),
    ])
    return _node_call(omsg[0], omsg[1], ocnt[0], ocnt[1], h, ntf,
                      U1[:, :D].T, U1[:, D:].T, U2.T, prm_e)


# speedup vs baseline: 7.7862x; 1.2910x over previous
import functools

import jax
import jax.numpy as jnp
from jax import lax
from jax.experimental import pallas as pl
from jax.experimental.pallas import tpu as pltpu
from jax.experimental.pallas import tpu_sc as plsc

N = 10000
E = 320000
D = 128
NUM_RBF = 32
CUTOFF = 6.0
STEP = CUTOFF / (NUM_RBF - 1)
GAMMA = 1.0 / max(STEP * STEP, 1e-06)

PW = 8
CW = 16
AW = 8
NBLK = 400
NB = N // NBLK
EBLK = 2560
EB = E // EBLK
NC = 2
NS = 16
NW = NC * NS
EPW = E // NW
CH = 80
NCH = EPW // CH
RPT = N // NS
STRIPE = 624

NSETS = 5
NBODY = NCH // NSETS
DCH = 40
DSETS = 2
DNCH = EPW // DCH
DNBODY = DNCH // DSETS


def _proj_body(h_ref, w1at_ref, w1bt_ref, w1ct_ref, emb8_ref,
               ha_ref, hb_ref, ec_ref):
    h = h_ref[...]
    ha_ref[...] = jnp.dot(h, w1at_ref[...], preferred_element_type=jnp.float32)
    hb_ref[...] = jnp.dot(h, w1bt_ref[...], preferred_element_type=jnp.float32)
    ec_ref[...] = jnp.dot(emb8_ref[...], w1ct_ref[...],
                          preferred_element_type=jnp.float32)


_proj_call = pl.pallas_call(
    _proj_body,
    grid=(NB,),
    in_specs=[
        pl.BlockSpec((NBLK, D), lambda i: (i, 0)),
        pl.BlockSpec((D, D), lambda i: (0, 0)),
        pl.BlockSpec((D, D), lambda i: (0, 0)),
        pl.BlockSpec((D, D), lambda i: (0, 0)),
        pl.BlockSpec((8, D), lambda i: (0, 0)),
    ],
    out_specs=[
        pl.BlockSpec((NBLK, D), lambda i: (i, 0)),
        pl.BlockSpec((NBLK, D), lambda i: (i, 0)),
        pl.BlockSpec((8, D), lambda i: (0, 0)),
    ],
    out_shape=[
        jax.ShapeDtypeStruct((N, D), jnp.float32),
        jax.ShapeDtypeStruct((N, D), jnp.float32),
        jax.ShapeDtypeStruct((8, D), jnp.float32),
    ],
)


@functools.cache
def _sc_mesh():
    return plsc.VectorSubcoreMesh(core_axis_name="c", subcore_axis_name="s")


@functools.cache
def _edge_gather_call():
    per_set = [
        pltpu.VMEM((CH,), jnp.int32),
        pltpu.VMEM((CH,), jnp.int32),
        pltpu.VMEM((CH, D), jnp.float32),
        pltpu.VMEM((CH, D), jnp.float32),
        pltpu.SemaphoreType.DMA,
    ]
    return pl.kernel(
        _edge_gather,
        out_type=jax.ShapeDtypeStruct((E, D), jnp.float32),
        mesh=_sc_mesh(),
        scratch_types=per_set * NSETS + [
            pltpu.SemaphoreType.DMA,
            pltpu.SemaphoreType.DMA,
        ],
    )


def _edge_gather(src_hbm, dst_hbm, tha_hbm, thb_hbm, g_hbm, *scr):
    sets = [scr[5 * s:5 * s + 5] for s in range(NSETS)]
    sem_i = scr[5 * NSETS]
    sem_o = scr[5 * NSETS + 1]
    wid = lax.axis_index("c") * NS + lax.axis_index("s")
    base = wid * EPW

    def drain_outs():
        for si_v, di_v, ra_v, rb_v, sem_g in sets:
            pltpu.make_async_copy(g_hbm.at[pl.ds(0, CH)], ra_v, sem_o).wait()

    def body(j, carry):
        @pl.when(j > 0)
        def _():
            drain_outs()

        offs = [pl.multiple_of(base + (j * NSETS + s) * CH, 8)
                for s in range(NSETS)]
        idx_cps = []
        for off, (si_v, di_v, *_rest) in zip(offs, sets):
            idx_cps.append(
                pltpu.async_copy(src_hbm.at[pl.ds(off, CH)], si_v, sem_i))
            idx_cps.append(
                pltpu.async_copy(dst_hbm.at[pl.ds(off, CH)], di_v, sem_i))
        for cp in idx_cps:
            cp.wait()
        gather_cps = []
        for si_v, di_v, ra_v, rb_v, sem_g in sets:
            gather_cps.append((
                pltpu.async_copy(tha_hbm.at[si_v], ra_v, sem_g),
                pltpu.async_copy(thb_hbm.at[di_v], rb_v, sem_g),
            ))
        for off, (si_v, di_v, ra_v, rb_v, sem_g), cps in zip(
                offs, sets, gather_cps):
            for cp in cps:
                cp.wait()

            def add4(r, c2, ra_v=ra_v, rb_v=rb_v):
                for rr in range(4):
                    row = r * 4 + rr
                    for k in range(D // 16):
                        sl = pl.ds(k * 16, 16)
                        ra_v[row, sl] = ra_v[row, sl] + rb_v[row, sl]
                return c2

            lax.fori_loop(0, CH // 4, add4, 0)
            pltpu.async_copy(ra_v, g_hbm.at[pl.ds(off, CH)], sem_o)
        return carry

    lax.fori_loop(0, NBODY, body, 0)
    drain_outs()


@functools.cache
def _aux_call():
    per_set = [
        pltpu.VMEM((CH,), jnp.int32),
        pltpu.VMEM((CH,), jnp.int32),
        pltpu.VMEM((CH,), jnp.float32),
        pltpu.VMEM((CH, PW), jnp.float32),
        pltpu.VMEM((CH, PW), jnp.float32),
        pltpu.VMEM((AW, CH), jnp.float32),
        pltpu.SemaphoreType.DMA,
        pltpu.SemaphoreType.DMA,
    ]
    return pl.kernel(
        _aux_kernel,
        out_type=(jax.ShapeDtypeStruct((AW, E), jnp.float32),
                  jax.ShapeDtypeStruct((NC, N, CW), jnp.float32)),
        mesh=_sc_mesh(),
        compiler_params=pltpu.CompilerParams(use_tc_tiling_on_sc=False,
                                             needs_layout_passes=False),
        scratch_types=per_set * NSETS + [
            pltpu.VMEM((CH, CW), jnp.float32),
            pltpu.VMEM_SHARED((N, CW), jnp.float32),
            pltpu.SemaphoreType.DMA,
            pltpu.SemaphoreType.DMA,
        ],
    )


def _aux_kernel(src_hbm, dst_hbm, etf_hbm, pos8_hbm, zcnt_hbm,
                relt_hbm, ocnt_hbm, *scr):
    sets = [scr[8 * s:8 * s + 8] for s in range(NSETS)]
    ones_v, cnt_sh, sem_i, sem_o = scr[8 * NSETS:]
    cid = lax.axis_index("c")
    sid = lax.axis_index("s")
    wid = cid * NS + sid
    rbase = sid * RPT
    pltpu.sync_copy(zcnt_hbm.at[pl.ds(rbase, RPT)],
                    cnt_sh.at[pl.ds(rbase, RPT)])
    onerow = jnp.where(lax.iota(jnp.int32, CW) == 0, 1.0, 0.0)

    def set_row(r, c):
        ones_v[r] = onerow
        return c

    lax.fori_loop(0, CH, set_row, 0)
    plsc.subcore_barrier()
    base = wid * EPW

    def drain_prev():
        for si_v, di_v, et_v, pa_v, pb_v, rt_v, sem_g, sem_s in sets:
            pltpu.make_async_copy(relt_hbm.at[:, pl.ds(0, CH)], rt_v,
                                  sem_o).wait()
            pltpu.make_async_copy(zcnt_hbm.at[pl.ds(0, CH)], ones_v,
                                  sem_s).wait()

    def body(j, carry):
        @pl.when(j > 0)
        def _():
            drain_prev()

        offs = [base + (j * NSETS + s) * CH for s in range(NSETS)]
        idx_cps = []
        for off, (si_v, di_v, et_v, *_r) in zip(offs, sets):
            idx_cps.append(
                pltpu.async_copy(src_hbm.at[pl.ds(off, CH)], si_v, sem_i))
            idx_cps.append(
                pltpu.async_copy(dst_hbm.at[pl.ds(off, CH)], di_v, sem_i))
            idx_cps.append(
                pltpu.async_copy(etf_hbm.at[pl.ds(off, CH)], et_v, sem_i))
        for cp in idx_cps:
            cp.wait()
        gather_cps = []
        for si_v, di_v, et_v, pa_v, pb_v, rt_v, sem_g, sem_s in sets:
            gather_cps.append((
                pltpu.async_copy(pos8_hbm.at[si_v], pa_v, sem_g),
                pltpu.async_copy(pos8_hbm.at[di_v], pb_v, sem_g),
            ))
        for off, (si_v, di_v, et_v, pa_v, pb_v, rt_v, sem_g, sem_s), cps in (
                zip(offs, sets, gather_cps)):
            for cp in cps:
                cp.wait()
            for k in range(CH // 16):
                sl = pl.ds(16 * k, 16)
                rows = lax.iota(jnp.int32, 16) + (16 * k)
                for c in range(3):
                    cols = jnp.full((16,), c, jnp.int32)
                    va = plsc.load_gather(pa_v, [rows, cols])
                    vb = plsc.load_gather(pb_v, [rows, cols])
                    rt_v[c, sl] = va - vb
                rt_v[3, sl] = et_v[sl]
            pltpu.async_copy(rt_v, relt_hbm.at[:, pl.ds(off, CH)], sem_o)
            pltpu.async_copy(ones_v, cnt_sh.at[di_v], sem_s, add=True)
        return carry

    lax.fori_loop(0, NBODY, body, 0)
    drain_prev()
    plsc.subcore_barrier()
    pltpu.sync_copy(cnt_sh.at[pl.ds(rbase, RPT)],
                    ocnt_hbm.at[cid, pl.ds(rbase, RPT)])


def _edge_mlp_body(g_ref, relt_ref, wr_ref, w2t_ref, prm_ref, out_ref):
    base = g_ref[...]
    aux = jnp.transpose(relt_ref[...])
    rel = aux[:, 0:3]
    t = aux[:, 3:4]
    dist2 = jnp.sum(rel * rel, axis=1, keepdims=True)
    dist = jnp.sqrt(dist2)
    centers = lax.broadcasted_iota(
        jnp.int32, (1, NUM_RBF), 1).astype(jnp.float32) * STEP
    diff = dist - centers
    radial = jnp.exp(-GAMMA * diff * diff)
    prm = prm_ref[...]
    bias = prm[0:1, :]
    wd = prm[1:2, :]
    we = prm[2:3, :]
    b2 = prm[3:4, :]
    pre = (base + bias + dist * wd + t * we
           + jnp.dot(radial, wr_ref[...], preferred_element_type=jnp.float32))
    x = pre * jax.nn.sigmoid(pre)
    m = jnp.dot(x, w2t_ref[...], preferred_element_type=jnp.float32) + b2
    out_ref[...] = m * jax.nn.sigmoid(m)


_edge_mlp_call = pl.pallas_call(
    _edge_mlp_body,
    grid=(EB,),
    in_specs=[
        pl.BlockSpec((EBLK, D), lambda i: (i, 0)),
        pl.BlockSpec((AW, EBLK), lambda i: (0, i)),
        pl.BlockSpec((NUM_RBF, D), lambda i: (0, 0)),
        pl.BlockSpec((D, D), lambda i: (0, 0)),
        pl.BlockSpec((8, D), lambda i: (0, 0)),
    ],
    out_specs=pl.BlockSpec((EBLK, D), lambda i: (i, 0)),
    out_shape=jax.ShapeDtypeStruct((E, D), jnp.float32),
)


@functools.cache
def _edge_scatter_call():
    return pl.kernel(
        _edge_scatter,
        out_type=jax.ShapeDtypeStruct((NC, N, D), jnp.float32),
        mesh=_sc_mesh(),
        scratch_types=[
            pltpu.VMEM((DCH,), jnp.int32),
            pltpu.VMEM((DCH, D), jnp.float32),
            pltpu.SemaphoreType.DMA,
        ] * DSETS + [
            pltpu.VMEM_SHARED((N, D), jnp.float32),
            pltpu.SemaphoreType.DMA,
        ],
    )


def _edge_scatter(dst_hbm, msg_hbm, zmsg_hbm, omsg_hbm, *scr):
    sets = [scr[3 * s:3 * s + 3] for s in range(DSETS)]
    acc_sh, sem_i = scr[3 * DSETS:]
    cid = lax.axis_index("c")
    sid = lax.axis_index("s")
    wid = cid * NS + sid
    rbase = pl.multiple_of(sid * STRIPE, 8)
    pltpu.sync_copy(zmsg_hbm.at[pl.ds(rbase, STRIPE)],
                    acc_sh.at[pl.ds(rbase, STRIPE)])

    @pl.when(sid == 0)
    def _():
        pltpu.sync_copy(zmsg_hbm.at[pl.ds(NS * STRIPE, N - NS * STRIPE)],
                        acc_sh.at[pl.ds(NS * STRIPE, N - NS * STRIPE)])

    plsc.subcore_barrier()
    base = wid * EPW

    def drain_set(buf_v, sem_sc):
        pltpu.make_async_copy(msg_hbm.at[pl.ds(0, DCH)], buf_v, sem_sc).wait()

    def body(j, carry):
        offs = [pl.multiple_of(base + (j * DSETS + s) * DCH, 8)
                for s in range(DSETS)]
        cps = []
        for off, (di_v, buf_v, sem_sc) in zip(offs, sets):
            @pl.when(j > 0)
            def _(buf_v=buf_v, sem_sc=sem_sc):
                drain_set(buf_v, sem_sc)

            cps.append((
                pltpu.async_copy(dst_hbm.at[pl.ds(off, DCH)], di_v, sem_i),
                pltpu.async_copy(msg_hbm.at[pl.ds(off, DCH)], buf_v, sem_i),
            ))
        for (cp_i, cp_m), (di_v, buf_v, sem_sc) in zip(cps, sets):
            cp_i.wait()
            cp_m.wait()
            pltpu.async_copy(buf_v, acc_sh.at[di_v], sem_sc, add=True)
        return carry

    lax.fori_loop(0, DNBODY, body, 0)
    for di_v, buf_v, sem_sc in sets:
        drain_set(buf_v, sem_sc)
    plsc.subcore_barrier()
    pltpu.sync_copy(acc_sh.at[pl.ds(rbase, STRIPE)],
                    omsg_hbm.at[cid, pl.ds(rbase, STRIPE)])

    @pl.when(sid == 0)
    def _():
        pltpu.sync_copy(acc_sh.at[pl.ds(NS * STRIPE, N - NS * STRIPE)],
                        omsg_hbm.at[cid, pl.ds(NS * STRIPE, N - NS * STRIPE)])


def _node_body(p0_ref, p1_ref, c0_ref, c1_ref, h_ref, ntf_ref,
               u1ht_ref, u1at_ref, u2t_ref, prm_ref, out_ref):
    cnt = jnp.maximum(c0_ref[:, :1] + c1_ref[:, :1], 1.0)
    agg = (p0_ref[...] + p1_ref[...]) / cnt
    h = h_ref[...]
    prm = prm_ref[...]
    c1row = prm[0:1, :]
    c2row = prm[1:2, :]
    gam = prm[2:3, :]
    bet = prm[3:4, :]
    upre = (jnp.dot(h, u1ht_ref[...], preferred_element_type=jnp.float32)
            + jnp.dot(agg, u1at_ref[...], preferred_element_type=jnp.float32)
            + c1row)
    u = upre * jax.nn.sigmoid(upre)
    upd = jnp.dot(u, u2t_ref[...], preferred_element_type=jnp.float32) + c2row
    pre = h + upd
    mu = jnp.mean(pre, axis=1, keepdims=True)
    cen = pre - mu
    var = jnp.mean(cen * cen, axis=1, keepdims=True)
    ln = cen / jnp.sqrt(var + 1e-05) * gam + bet
    out_ref[...] = jnp.where(ntf_ref[...] == 1.0, ln, h)


_node_call = pl.pallas_call(
    _node_body,
    grid=(NB,),
    in_specs=[
        pl.BlockSpec((NBLK, D), lambda i: (i, 0)),
        pl.BlockSpec((NBLK, D), lambda i: (i, 0)),
        pl.BlockSpec((NBLK, CW), lambda i: (i, 0)),
        pl.BlockSpec((NBLK, CW), lambda i: (i, 0)),
        pl.BlockSpec((NBLK, D), lambda i: (i, 0)),
        pl.BlockSpec((NBLK, 1), lambda i: (i, 0)),
        pl.BlockSpec((D, D), lambda i: (0, 0)),
        pl.BlockSpec((D, D), lambda i: (0, 0)),
        pl.BlockSpec((D, D), lambda i: (0, 0)),
        pl.BlockSpec((8, D), lambda i: (0, 0)),
    ],
    out_specs=pl.BlockSpec((NBLK, D), lambda i: (i, 0)),
    out_shape=jax.ShapeDtypeStruct((N, D), jnp.float32),
)


def kernel(h, pos, edge_index, edge_type, node_type, emb, W1, b1, W2, b2,
           U1, c1, U2, c2, gamma_ln, beta_ln):
    src = edge_index[0]
    dst = edge_index[1]
    etf = edge_type.astype(jnp.float32)
    ntf = node_type.astype(jnp.float32).reshape(N, 1)

    w1at = W1[:, :D].T
    w1bt = W1[:, D:2 * D].T
    w1ct = W1[:, 2 * D:3 * D].T
    wr = W1[:, 3 * D:3 * D + NUM_RBF].T
    wd = W1[:, 3 * D + NUM_RBF]
    emb8 = jnp.concatenate([emb, jnp.zeros((8 - emb.shape[0], D), jnp.float32)])

    ha, hb, ec28 = _proj_call(h, w1at, w1bt, w1ct, emb8)
    ec0 = ec28[0]
    dec = ec28[1] - ec28[0]
    prm_c = jnp.concatenate([
        jnp.stack([b1 + ec0, wd, dec, b2]),
        jnp.zeros((4, D), jnp.float32),
    ])
    pos8 = jnp.concatenate([pos, jnp.zeros((N, PW - 3), jnp.float32)], axis=1)

    g = _edge_gather_call()(src, dst, ha, hb)
    zcnt = jnp.zeros((N, CW), jnp.float32)
    relt, ocnt = _aux_call()(src, dst, etf, pos8, zcnt)
    msg = _edge_mlp_call(g, relt, wr, W2.T, prm_c)

    zmsg = jnp.zeros((N, D), jnp.float32)
    omsg = _edge_scatter_call()(dst, msg, zmsg)

    prm_e = jnp.concatenate([
        jnp.stack([c1, c2, gamma_ln, beta_ln]),
        jnp.zeros((4, D), jnp.float32),
    ])
    return _node_call(omsg[0], omsg[1], ocnt[0], ocnt[1], h, ntf,
                      U1[:, :D].T, U1[:, D:].T, U2.T, prm_e)
